# baseline (speedup 1.0000x reference)
import functools

import jax
import jax.numpy as jnp
from jax import lax
from jax.experimental import pallas as pl
from jax.experimental.pallas import tpu as pltpu
from jax.experimental.pallas import tpu_sc as plsc

TOKENS = 32768
D_MODEL = 4096
N_MODELS = 64
TOPK = 8
BLOCK_T = 512

SLICES = 8
TS = TOKENS // SLICES
N_WORKERS = 32
T_PER_W = TS // N_WORKERS
LANES = 16
NBLK = 4


def _dense_body(x_ref, w_ref, nz_ref, o_ref):
    acc = jnp.dot(
        x_ref[:].astype(jnp.bfloat16),
        w_ref[:],
        preferred_element_type=jnp.float32,
    )
    hg = acc[:, :N_MODELS]
    sp = acc[:, N_MODELS:]
    h = hg + nz_ref[:] * jnp.logaddexp(sp, 0.0)
    o_ref[:] = h.T


def _dense(x, w_cat, noise, s):
    base = s * (TS // BLOCK_T)
    return pl.pallas_call(
        _dense_body,
        grid=(TS // BLOCK_T,),
        in_specs=[
            pl.BlockSpec((BLOCK_T, D_MODEL), lambda i: (base + i, 0)),
            pl.BlockSpec((D_MODEL, 2 * N_MODELS), lambda i: (0, 0)),
            pl.BlockSpec((BLOCK_T, N_MODELS), lambda i: (base + i, 0)),
        ],
        out_specs=pl.BlockSpec((N_MODELS, BLOCK_T), lambda i: (0, i)),
        out_shape=jax.ShapeDtypeStruct((N_MODELS, TS), jnp.float32),
        compiler_params=pltpu.CompilerParams(
            dimension_semantics=("arbitrary",),
        ),
    )(x, w_cat, noise)


def _ce(t, i, j):
    hi = jnp.maximum(t[i], t[j])
    lo = jnp.minimum(t[i], t[j])
    t[i], t[j] = hi, lo


def _merge_top8(a, b):
    c = [jnp.maximum(a[i], b[TOPK - 1 - i]) for i in range(TOPK)]
    for i in range(4):
        _ce(c, i, i + 4)
    for base in (0, 4):
        _ce(c, base, base + 2)
        _ce(c, base + 1, base + 3)
    for i in (0, 2, 4, 6):
        _ce(c, i, i + 1)
    return c


def _sc_route_body(ht_hbm, g_hbm, in_v, out_v, sem):
    wid = lax.axis_index("s") * 2 + lax.axis_index("c")
    t0w = wid * T_PER_W
    lane = lax.iota(jnp.int32, LANES)
    neg_inf = jnp.full((LANES,), -jnp.inf, jnp.float32)

    copies = [
        pltpu.async_copy(
            ht_hbm.at[pl.ds(e * TS + t0w, T_PER_W)],
            in_v.at[pl.ds(e * T_PER_W, T_PER_W)],
            sem,
        )
        for e in range(N_MODELS)
    ]
    for c in copies:
        c.wait()

    eb = N_MODELS // NBLK

    def group_body(g, carry2):
        gbase = g * LANES
        blocks = []
        for b in range(NBLK):
            t = [neg_inf] * TOPK
            for k in range(eb):
                e = b * eb + k
                x = in_v[pl.ds(e * T_PER_W + gbase, LANES)]
                for j in range(TOPK):
                    hi = jnp.maximum(t[j], x)
                    x = jnp.minimum(t[j], x)
                    t[j] = hi
            blocks.append(t)
        m01 = _merge_top8(blocks[0], blocks[1])
        m23 = _merge_top8(blocks[2], blocks[3])
        m = [jnp.maximum(m01[i], m23[TOPK - 1 - i]) for i in range(TOPK)]
        row_max = m[0]
        kth = m[0]
        for i in range(1, TOPK):
            row_max = jnp.maximum(row_max, m[i])
            kth = jnp.minimum(kth, m[i])
        s = jnp.zeros((LANES,), jnp.float32)
        for i in range(TOPK):
            s = s + jnp.exp(m[i] - row_max)
        inv = 1.0 / s
        rowbase = gbase * N_MODELS + lane * N_MODELS
        for e in range(N_MODELS):
            col = lane + e
            col = jnp.where(col >= N_MODELS, col - N_MODELS, col)
            x = plsc.load_gather(in_v, [col * T_PER_W + gbase + lane])
            v = jnp.where(x >= kth, jnp.exp(x - row_max) * inv, 0.0)
            plsc.store_scatter(out_v, [rowbase + col], v)
        return carry2

    lax.fori_loop(0, T_PER_W // LANES, group_body, 0)
    pltpu.sync_copy(
        out_v,
        g_hbm.at[pl.ds(t0w * N_MODELS, T_PER_W * N_MODELS)],
    )


_sc_route = functools.partial(
    pl.kernel,
    mesh=plsc.VectorSubcoreMesh(core_axis_name="c", subcore_axis_name="s"),
    out_type=jax.ShapeDtypeStruct((TS * N_MODELS,), jnp.float32),
    scratch_types=[
        pltpu.VMEM((N_MODELS * T_PER_W,), jnp.float32),
        pltpu.VMEM((T_PER_W * N_MODELS,), jnp.float32),
        pltpu.SemaphoreType.DMA,
    ],
    compiler_params=pltpu.CompilerParams(needs_layout_passes=False),
)(_sc_route_body)


def kernel(noise_key, x, W_g, W_noise):
    x2 = x if x.ndim == 2 else x.reshape((x.shape[0], -1))
    noise = jax.random.normal(noise_key, shape=(x2.shape[0], N_MODELS))
    w_cat = jnp.concatenate([W_g, W_noise], axis=1).astype(jnp.bfloat16)
    gs = []
    for s in range(SLICES):
        h_t = _dense(x2, w_cat, noise, s)
        gs.append(_sc_route(h_t.reshape((N_MODELS * TS,))))
    g = jnp.concatenate(gs)
    return g.reshape((TOKENS, N_MODELS))


# speedup vs baseline: 1.0497x; 1.0497x over previous
import functools

import jax
import jax.numpy as jnp
from jax import lax
from jax.experimental import pallas as pl
from jax.experimental.pallas import tpu as pltpu
from jax.experimental.pallas import tpu_sc as plsc

TOKENS = 32768
D_MODEL = 4096
N_MODELS = 64
TOPK = 8
BLOCK_T = 512

SLICES = 2
TS = TOKENS // SLICES
N_WORKERS = 32
T_PER_W = TS // N_WORKERS
LANES = 16
NBLK = 4


def _dense_body(x_ref, w_ref, nz_ref, o_ref):
    acc = jnp.dot(
        x_ref[:].astype(jnp.bfloat16),
        w_ref[:],
        preferred_element_type=jnp.float32,
    )
    hg = acc[:, :N_MODELS]
    sp = acc[:, N_MODELS:]
    h = hg + nz_ref[:] * jnp.logaddexp(sp, 0.0)
    o_ref[:] = h.T


def _dense(x, w_cat, noise, s):
    base = s * (TS // BLOCK_T)
    return pl.pallas_call(
        _dense_body,
        grid=(TS // BLOCK_T,),
        in_specs=[
            pl.BlockSpec((BLOCK_T, D_MODEL), lambda i: (base + i, 0)),
            pl.BlockSpec((D_MODEL, 2 * N_MODELS), lambda i: (0, 0)),
            pl.BlockSpec((BLOCK_T, N_MODELS), lambda i: (base + i, 0)),
        ],
        out_specs=pl.BlockSpec((N_MODELS, BLOCK_T), lambda i: (0, i)),
        out_shape=jax.ShapeDtypeStruct((N_MODELS, TS), jnp.float32),
        compiler_params=pltpu.CompilerParams(
            dimension_semantics=("arbitrary",),
        ),
    )(x, w_cat, noise)


def _ce(t, i, j):
    hi = jnp.maximum(t[i], t[j])
    lo = jnp.minimum(t[i], t[j])
    t[i], t[j] = hi, lo


def _merge_top8(a, b):
    c = [jnp.maximum(a[i], b[TOPK - 1 - i]) for i in range(TOPK)]
    for i in range(4):
        _ce(c, i, i + 4)
    for base in (0, 4):
        _ce(c, base, base + 2)
        _ce(c, base + 1, base + 3)
    for i in (0, 2, 4, 6):
        _ce(c, i, i + 1)
    return c


def _sc_route_body(ht_hbm, g_hbm, in_v, out_v, sem):
    wid = lax.axis_index("s") * 2 + lax.axis_index("c")
    t0w = wid * T_PER_W
    lane = lax.iota(jnp.int32, LANES)
    neg_inf = jnp.full((LANES,), -jnp.inf, jnp.float32)

    copies = [
        pltpu.async_copy(
            ht_hbm.at[pl.ds(e * TS + t0w, T_PER_W)],
            in_v.at[pl.ds(e * T_PER_W, T_PER_W)],
            sem,
        )
        for e in range(N_MODELS)
    ]
    for c in copies:
        c.wait()

    eb = N_MODELS // NBLK

    def group_body(g, carry2):
        gbase = g * LANES
        blocks = []
        for b in range(NBLK):
            t = [neg_inf] * TOPK
            for k in range(eb):
                e = b * eb + k
                x = in_v[pl.ds(e * T_PER_W + gbase, LANES)]
                for j in range(TOPK):
                    hi = jnp.maximum(t[j], x)
                    x = jnp.minimum(t[j], x)
                    t[j] = hi
            blocks.append(t)
        m01 = _merge_top8(blocks[0], blocks[1])
        m23 = _merge_top8(blocks[2], blocks[3])
        m = [jnp.maximum(m01[i], m23[TOPK - 1 - i]) for i in range(TOPK)]
        row_max = m[0]
        kth = m[0]
        for i in range(1, TOPK):
            row_max = jnp.maximum(row_max, m[i])
            kth = jnp.minimum(kth, m[i])
        s = jnp.zeros((LANES,), jnp.float32)
        for i in range(TOPK):
            s = s + jnp.exp(m[i] - row_max)
        inv = 1.0 / s
        rowbase = gbase * N_MODELS + lane * N_MODELS
        for e in range(N_MODELS):
            col = lane + e
            col = jnp.where(col >= N_MODELS, col - N_MODELS, col)
            x = plsc.load_gather(in_v, [col * T_PER_W + gbase + lane])
            v = jnp.where(x >= kth, jnp.exp(x - row_max) * inv, 0.0)
            plsc.store_scatter(out_v, [rowbase + col], v)
        return carry2

    lax.fori_loop(0, T_PER_W // LANES, group_body, 0)
    pltpu.sync_copy(
        out_v,
        g_hbm.at[pl.ds(t0w * N_MODELS, T_PER_W * N_MODELS)],
    )


_sc_route = functools.partial(
    pl.kernel,
    mesh=plsc.VectorSubcoreMesh(core_axis_name="c", subcore_axis_name="s"),
    out_type=jax.ShapeDtypeStruct((TS * N_MODELS,), jnp.float32),
    scratch_types=[
        pltpu.VMEM((N_MODELS * T_PER_W,), jnp.float32),
        pltpu.VMEM((T_PER_W * N_MODELS,), jnp.float32),
        pltpu.SemaphoreType.DMA,
    ],
    compiler_params=pltpu.CompilerParams(needs_layout_passes=False),
)(_sc_route_body)


def kernel(noise_key, x, W_g, W_noise):
    x2 = x if x.ndim == 2 else x.reshape((x.shape[0], -1))
    noise = jax.random.normal(noise_key, shape=(x2.shape[0], N_MODELS))
    w_cat = jnp.concatenate([W_g, W_noise], axis=1).astype(jnp.bfloat16)
    gs = []
    for s in range(SLICES):
        h_t = _dense(x2, w_cat, noise, s)
        gs.append(_sc_route(h_t.reshape((N_MODELS * TS,))))
    g = jnp.concatenate(gs)
    return g.reshape((TOKENS, N_MODELS))


# speedup vs baseline: 1.0942x; 1.0424x over previous
import functools

import jax
import jax.numpy as jnp
from jax import lax
from jax.experimental import pallas as pl
from jax.experimental.pallas import tpu as pltpu
from jax.experimental.pallas import tpu_sc as plsc

TOKENS = 32768
D_MODEL = 4096
N_MODELS = 64
TOPK = 8
BLOCK_T = 512

SLICES = 2
TS = TOKENS // SLICES
N_WORKERS = 32
T_PER_W = TS // N_WORKERS
LANES = 16
NBLK = 4


def _dense_body(x_ref, w_ref, nz_ref, o_ref):
    acc = jnp.dot(
        x_ref[:].astype(jnp.bfloat16),
        w_ref[:],
        preferred_element_type=jnp.float32,
    )
    hg = acc[:, :N_MODELS]
    sp = acc[:, N_MODELS:]
    h = hg + nz_ref[:] * jnp.logaddexp(sp, 0.0)
    o_ref[:] = h.T


def _dense(x, w_cat, noise, s):
    base = s * (TS // BLOCK_T)
    return pl.pallas_call(
        _dense_body,
        grid=(TS // BLOCK_T,),
        in_specs=[
            pl.BlockSpec((BLOCK_T, D_MODEL), lambda i: (base + i, 0)),
            pl.BlockSpec((D_MODEL, 2 * N_MODELS), lambda i: (0, 0)),
            pl.BlockSpec((BLOCK_T, N_MODELS), lambda i: (base + i, 0)),
        ],
        out_specs=pl.BlockSpec((N_MODELS, BLOCK_T), lambda i: (0, i)),
        out_shape=jax.ShapeDtypeStruct((N_MODELS, TS), jnp.float32),
        compiler_params=pltpu.CompilerParams(
            dimension_semantics=("arbitrary",),
        ),
    )(x, w_cat, noise)


def _ce(t, i, j):
    hi = jnp.maximum(t[i], t[j])
    lo = jnp.minimum(t[i], t[j])
    t[i], t[j] = hi, lo


def _merge_top8(a, b):
    c = [jnp.maximum(a[i], b[TOPK - 1 - i]) for i in range(TOPK)]
    for i in range(4):
        _ce(c, i, i + 4)
    for base in (0, 4):
        _ce(c, base, base + 2)
        _ce(c, base + 1, base + 3)
    for i in (0, 2, 4, 6):
        _ce(c, i, i + 1)
    return c


def _sc_route_body(ht_hbm, g_hbm, in_v, out_v, sem):
    wid = lax.axis_index("s") * 2 + lax.axis_index("c")
    t0w = wid * T_PER_W
    lane = lax.iota(jnp.int32, LANES)
    neg_inf = jnp.full((LANES,), -jnp.inf, jnp.float32)

    copies = [
        pltpu.async_copy(
            ht_hbm.at[e, pl.ds(t0w, T_PER_W)],
            in_v.at[pl.ds(e * T_PER_W, T_PER_W)],
            sem,
        )
        for e in range(N_MODELS)
    ]
    for c in copies:
        c.wait()

    eb = N_MODELS // NBLK

    def group_body(g, carry2):
        gbase = g * LANES
        blocks = []
        for b in range(NBLK):
            t = [neg_inf] * TOPK
            for k in range(eb):
                e = b * eb + k
                x = in_v[pl.ds(e * T_PER_W + gbase, LANES)]
                for j in range(TOPK):
                    hi = jnp.maximum(t[j], x)
                    x = jnp.minimum(t[j], x)
                    t[j] = hi
            blocks.append(t)
        m01 = _merge_top8(blocks[0], blocks[1])
        m23 = _merge_top8(blocks[2], blocks[3])
        m = [jnp.maximum(m01[i], m23[TOPK - 1 - i]) for i in range(TOPK)]
        row_max = m[0]
        kth = m[0]
        for i in range(1, TOPK):
            row_max = jnp.maximum(row_max, m[i])
            kth = jnp.minimum(kth, m[i])
        s = jnp.zeros((LANES,), jnp.float32)
        for i in range(TOPK):
            s = s + jnp.exp(m[i] - row_max)
        inv = 1.0 / s
        rows = gbase + lane
        for e in range(N_MODELS):
            col = lane + e
            col = jnp.where(col >= N_MODELS, col - N_MODELS, col)
            x = plsc.load_gather(in_v, [col * T_PER_W + gbase + lane])
            v = jnp.where(x >= kth, jnp.exp(x - row_max) * inv, 0.0)
            plsc.store_scatter(out_v, [rows, col], v)
        return carry2

    lax.fori_loop(0, T_PER_W // LANES, group_body, 0)
    pltpu.sync_copy(out_v, g_hbm.at[pl.ds(t0w, T_PER_W)])


_sc_route = functools.partial(
    pl.kernel,
    mesh=plsc.VectorSubcoreMesh(core_axis_name="c", subcore_axis_name="s"),
    out_type=jax.ShapeDtypeStruct((TS, N_MODELS), jnp.float32),
    scratch_types=[
        pltpu.VMEM((N_MODELS * T_PER_W,), jnp.float32),
        pltpu.VMEM((T_PER_W, N_MODELS), jnp.float32),
        pltpu.SemaphoreType.DMA,
    ],
    compiler_params=pltpu.CompilerParams(needs_layout_passes=False),
)(_sc_route_body)


def kernel(noise_key, x, W_g, W_noise):
    x2 = x if x.ndim == 2 else x.reshape((x.shape[0], -1))
    noise = jax.random.normal(noise_key, shape=(x2.shape[0], N_MODELS))
    w_cat = jnp.concatenate([W_g, W_noise], axis=1).astype(jnp.bfloat16)
    gs = []
    for s in range(SLICES):
        h_t = _dense(x2, w_cat, noise, s)
        gs.append(_sc_route(h_t))
    return jnp.concatenate(gs, axis=0)


# speedup vs baseline: 1.0997x; 1.0050x over previous
import functools

import jax
import jax.numpy as jnp
from jax import lax
from jax.experimental import pallas as pl
from jax.experimental.pallas import tpu as pltpu
from jax.experimental.pallas import tpu_sc as plsc

TOKENS = 32768
D_MODEL = 4096
N_MODELS = 64
TOPK = 8
BLOCK_T = 512

SLICES = 4
TS = TOKENS // SLICES
N_WORKERS = 32
T_PER_W = TS // N_WORKERS
LANES = 16
NBLK = 4


def _dense_body(x_ref, w_ref, nz_ref, o_ref):
    acc = jnp.dot(
        x_ref[:].astype(jnp.bfloat16),
        w_ref[:],
        preferred_element_type=jnp.float32,
    )
    hg = acc[:, :N_MODELS]
    sp = acc[:, N_MODELS:]
    h = hg + nz_ref[:] * jnp.logaddexp(sp, 0.0)
    o_ref[:] = h.T


def _dense(x, w_cat, noise, s):
    base = s * (TS // BLOCK_T)
    return pl.pallas_call(
        _dense_body,
        grid=(TS // BLOCK_T,),
        in_specs=[
            pl.BlockSpec((BLOCK_T, D_MODEL), lambda i: (base + i, 0)),
            pl.BlockSpec((D_MODEL, 2 * N_MODELS), lambda i: (0, 0)),
            pl.BlockSpec((BLOCK_T, N_MODELS), lambda i: (base + i, 0)),
        ],
        out_specs=pl.BlockSpec((N_MODELS, BLOCK_T), lambda i: (0, i)),
        out_shape=jax.ShapeDtypeStruct((N_MODELS, TS), jnp.float32),
        compiler_params=pltpu.CompilerParams(
            dimension_semantics=("arbitrary",),
        ),
    )(x, w_cat, noise)


def _ce(t, i, j):
    hi = jnp.maximum(t[i], t[j])
    lo = jnp.minimum(t[i], t[j])
    t[i], t[j] = hi, lo


def _merge_top8(a, b):
    c = [jnp.maximum(a[i], b[TOPK - 1 - i]) for i in range(TOPK)]
    for i in range(4):
        _ce(c, i, i + 4)
    for base in (0, 4):
        _ce(c, base, base + 2)
        _ce(c, base + 1, base + 3)
    for i in (0, 2, 4, 6):
        _ce(c, i, i + 1)
    return c


def _sc_route_body(ht_hbm, g_hbm, in_v, out_v, sem):
    wid = lax.axis_index("s") * 2 + lax.axis_index("c")
    t0w = wid * T_PER_W
    lane = lax.iota(jnp.int32, LANES)
    neg_inf = jnp.full((LANES,), -jnp.inf, jnp.float32)

    copies = [
        pltpu.async_copy(
            ht_hbm.at[e, pl.ds(t0w, T_PER_W)],
            in_v.at[pl.ds(e * T_PER_W, T_PER_W)],
            sem,
        )
        for e in range(N_MODELS)
    ]
    for c in copies:
        c.wait()

    eb = N_MODELS // NBLK

    def group_body(g, carry2):
        gbase = g * LANES
        blocks = []
        for b in range(NBLK):
            t = [neg_inf] * TOPK
            for k in range(eb):
                e = b * eb + k
                x = in_v[pl.ds(e * T_PER_W + gbase, LANES)]
                for j in range(TOPK):
                    hi = jnp.maximum(t[j], x)
                    x = jnp.minimum(t[j], x)
                    t[j] = hi
            blocks.append(t)
        m01 = _merge_top8(blocks[0], blocks[1])
        m23 = _merge_top8(blocks[2], blocks[3])
        m = [jnp.maximum(m01[i], m23[TOPK - 1 - i]) for i in range(TOPK)]
        row_max = m[0]
        kth = m[0]
        for i in range(1, TOPK):
            row_max = jnp.maximum(row_max, m[i])
            kth = jnp.minimum(kth, m[i])
        s = jnp.zeros((LANES,), jnp.float32)
        for i in range(TOPK):
            s = s + jnp.exp(m[i] - row_max)
        inv = 1.0 / s
        rows = gbase + lane
        for e in range(N_MODELS):
            col = lane + e
            col = jnp.where(col >= N_MODELS, col - N_MODELS, col)
            x = plsc.load_gather(in_v, [col * T_PER_W + gbase + lane])
            v = jnp.where(x >= kth, jnp.exp(x - row_max) * inv, 0.0)
            plsc.store_scatter(out_v, [rows, col], v)
        return carry2

    lax.fori_loop(0, T_PER_W // LANES, group_body, 0)
    pltpu.sync_copy(out_v, g_hbm.at[pl.ds(t0w, T_PER_W)])


_sc_route = functools.partial(
    pl.kernel,
    mesh=plsc.VectorSubcoreMesh(core_axis_name="c", subcore_axis_name="s"),
    out_type=jax.ShapeDtypeStruct((TS, N_MODELS), jnp.float32),
    scratch_types=[
        pltpu.VMEM((N_MODELS * T_PER_W,), jnp.float32),
        pltpu.VMEM((T_PER_W, N_MODELS), jnp.float32),
        pltpu.SemaphoreType.DMA,
    ],
    compiler_params=pltpu.CompilerParams(needs_layout_passes=False),
)(_sc_route_body)


def kernel(noise_key, x, W_g, W_noise):
    x2 = x if x.ndim == 2 else x.reshape((x.shape[0], -1))
    noise = jax.random.normal(noise_key, shape=(x2.shape[0], N_MODELS))
    w_cat = jnp.concatenate([W_g, W_noise], axis=1).astype(jnp.bfloat16)
    gs = []
    for s in range(SLICES):
        h_t = _dense(x2, w_cat, noise, s)
        gs.append(_sc_route(h_t))
    return jnp.concatenate(gs, axis=0)


# speedup vs baseline: 1.1008x; 1.0010x over previous
import functools

import jax
import jax.numpy as jnp
from jax import lax
from jax.experimental import pallas as pl
from jax.experimental.pallas import tpu as pltpu
from jax.experimental.pallas import tpu_sc as plsc

TOKENS = 32768
D_MODEL = 4096
N_MODELS = 64
TOPK = 8
BLOCK_T = 512

SLICES = 4
TS = TOKENS // SLICES
N_WORKERS = 32
T_PER_W = TS // N_WORKERS
LANES = 16
NBLK = 4


def _dense_body(x_ref, w_ref, nz_ref, o_ref):
    acc = jnp.dot(
        x_ref[:].astype(jnp.bfloat16),
        w_ref[:],
        preferred_element_type=jnp.float32,
    )
    hg = acc[:, :N_MODELS]
    sp = acc[:, N_MODELS:]
    h = hg + nz_ref[:] * jnp.logaddexp(sp, 0.0)
    o_ref[:] = h.T


def _dense(x, w_cat, noise, s):
    base = s * (TS // BLOCK_T)
    return pl.pallas_call(
        _dense_body,
        grid=(TS // BLOCK_T,),
        in_specs=[
            pl.BlockSpec((BLOCK_T, D_MODEL), lambda i: (base + i, 0)),
            pl.BlockSpec((D_MODEL, 2 * N_MODELS), lambda i: (0, 0)),
            pl.BlockSpec((BLOCK_T, N_MODELS), lambda i: (base + i, 0)),
        ],
        out_specs=pl.BlockSpec((N_MODELS, BLOCK_T), lambda i: (0, i)),
        out_shape=jax.ShapeDtypeStruct((N_MODELS, TS), jnp.float32),
        compiler_params=pltpu.CompilerParams(
            dimension_semantics=("arbitrary",),
        ),
    )(x, w_cat, noise)


def _ce(t, i, j):
    hi = jnp.maximum(t[i], t[j])
    lo = jnp.minimum(t[i], t[j])
    t[i], t[j] = hi, lo


def _merge_top8(a, b):
    c = [jnp.maximum(a[i], b[TOPK - 1 - i]) for i in range(TOPK)]
    for i in range(4):
        _ce(c, i, i + 4)
    for base in (0, 4):
        _ce(c, base, base + 2)
        _ce(c, base + 1, base + 3)
    for i in (0, 2, 4, 6):
        _ce(c, i, i + 1)
    return c


def _sc_route_body(ht_hbm, g_hbm, in_v, out_v, sem):
    wid = lax.axis_index("s") * 2 + lax.axis_index("c")
    t0w = wid * T_PER_W
    lane = lax.iota(jnp.int32, LANES)
    neg_inf = jnp.full((LANES,), -jnp.inf, jnp.float32)

    copies = [
        pltpu.async_copy(
            ht_hbm.at[e, pl.ds(t0w, T_PER_W)],
            in_v.at[pl.ds(e * T_PER_W, T_PER_W)],
            sem,
        )
        for e in range(N_MODELS)
    ]
    for c in copies:
        c.wait()

    eb = N_MODELS // NBLK

    @plsc.parallel_loop(0, T_PER_W // LANES, 1)
    def group_body(g):
        gbase = g * LANES
        blocks = []
        for b in range(NBLK):
            t = [neg_inf] * TOPK
            for k in range(eb):
                e = b * eb + k
                x = in_v[pl.ds(e * T_PER_W + gbase, LANES)]
                for j in range(TOPK):
                    hi = jnp.maximum(t[j], x)
                    x = jnp.minimum(t[j], x)
                    t[j] = hi
            blocks.append(t)
        m01 = _merge_top8(blocks[0], blocks[1])
        m23 = _merge_top8(blocks[2], blocks[3])
        m = [jnp.maximum(m01[i], m23[TOPK - 1 - i]) for i in range(TOPK)]
        row_max = m[0]
        kth = m[0]
        for i in range(1, TOPK):
            row_max = jnp.maximum(row_max, m[i])
            kth = jnp.minimum(kth, m[i])
        s = jnp.zeros((LANES,), jnp.float32)
        for i in range(TOPK):
            s = s + jnp.exp(m[i] - row_max)
        inv = 1.0 / s
        rows = gbase + lane
        for e in range(N_MODELS):
            col = lane + e
            col = jnp.where(col >= N_MODELS, col - N_MODELS, col)
            x = plsc.load_gather(in_v, [col * T_PER_W + gbase + lane])
            v = jnp.where(x >= kth, jnp.exp(x - row_max) * inv, 0.0)
            plsc.store_scatter(out_v, [rows, col], v)

    pltpu.sync_copy(out_v, g_hbm.at[pl.ds(t0w, T_PER_W)])


_sc_route = functools.partial(
    pl.kernel,
    mesh=plsc.VectorSubcoreMesh(core_axis_name="c", subcore_axis_name="s"),
    out_type=jax.ShapeDtypeStruct((TS, N_MODELS), jnp.float32),
    scratch_types=[
        pltpu.VMEM((N_MODELS * T_PER_W,), jnp.float32),
        pltpu.VMEM((T_PER_W, N_MODELS), jnp.float32),
        pltpu.SemaphoreType.DMA,
    ],
    compiler_params=pltpu.CompilerParams(needs_layout_passes=False),
)(_sc_route_body)


def kernel(noise_key, x, W_g, W_noise):
    x2 = x if x.ndim == 2 else x.reshape((x.shape[0], -1))
    noise = jax.random.normal(noise_key, shape=(x2.shape[0], N_MODELS))
    w_cat = jnp.concatenate([W_g, W_noise], axis=1).astype(jnp.bfloat16)
    gs = []
    for s in range(SLICES):
        h_t = _dense(x2, w_cat, noise, s)
        gs.append(_sc_route(h_t))
    return jnp.concatenate(gs, axis=0)
